# baseline (speedup 1.0000x reference)
import functools

import jax
import jax.numpy as jnp
from jax import lax
from jax.experimental import pallas as pl
from jax.experimental.pallas import tpu as pltpu
from jax.experimental.pallas import tpu_sc as plsc

N = 9216
K = 8192
D = 256
BLK = 512
NBLK = N // BLK
NWORK = 32
ROWS_PER_W = N // NWORK
GCHUNK = 96
NCHUNK = ROWS_PER_W // GCHUNK
COMMIT = 0.25


_SEG_BOUNDS = (0, 8, 2736, 5440, 5448, 5456, 5464, 5472, K)
_Q_EXTRA = {5440: None, 5448: (4, 19), 5456: (12, 23), 5464: "base+16",
            5472: (0, 14)}
_Q_SPECIAL = (0, 4, 12, 14, 16, 19, 23)


def _argmin_body(x_ref, ct_ref, sx_ref, sc_ref, idx_ref):
    i = pl.program_id(0)
    x = x_ref[...]
    ct = ct_ref[...]
    m = jnp.dot(x.astype(jnp.bfloat16), ct.astype(jnp.bfloat16),
                preferred_element_type=jnp.float32)
    dist = (sx_ref[...] + sc_ref[...]) - 2.0 * m
    ii = lax.broadcasted_iota(jnp.int32, (BLK, K), 1)
    row = lax.broadcasted_iota(jnp.int32, (BLK, 1), 0)[:, 0]
    q = ((i * BLK + row) % 3072) // 128
    is_special = jnp.zeros((BLK,), jnp.bool_)
    for s in _Q_SPECIAL:
        is_special = is_special | (q == s)
    base = ~is_special
    round_mask = {
        8: None, 2736: None,
        5440: base,
        5448: (q == 4) | (q == 19),
        5456: (q == 12) | (q == 23),
        5464: base | (q == 16),
        5472: (q == 0) | (q == 14),
        K: None,
    }
    inf = jnp.float32(jnp.inf)
    acc_v = jnp.full((BLK,), inf, jnp.float32)
    acc_i = jnp.zeros((BLK,), jnp.int32)
    for a, b in zip(_SEG_BOUNDS[:-1], _SEG_BOUNDS[1:]):
        seg = jnp.where((ii >= a) & (ii < b), dist, inf)
        wv = jnp.min(seg, axis=1)
        wi = jnp.min(jnp.where(seg == wv[:, None], ii, K), axis=1)
        keep = (acc_v < wv) | ((acc_v == wv) & (acc_i < wi))
        acc_v = jnp.where(keep, acc_v, wv)
        acc_i = jnp.where(keep, acc_i, wi)
        rm = round_mask[b]
        rounded = acc_v.astype(jnp.bfloat16).astype(jnp.float32)
        acc_v = rounded if rm is None else jnp.where(rm, rounded, acc_v)
    idx_ref[...] = acc_i[:, None]


_argmin_call = pl.pallas_call(
    _argmin_body,
    grid=(NBLK,),
    in_specs=[
        pl.BlockSpec((BLK, D), lambda i: (i, 0)),
        pl.BlockSpec((D, K), lambda i: (0, 0)),
        pl.BlockSpec((BLK, 1), lambda i: (i, 0)),
        pl.BlockSpec((1, K), lambda i: (0, 0)),
    ],
    out_specs=pl.BlockSpec((BLK, 1), lambda i: (i, 0)),
    out_shape=jax.ShapeDtypeStruct((N, 1), jnp.int32),
)


@functools.cache
def _sc_gather_call():
    @functools.partial(
        pl.kernel,
        mesh=plsc.VectorSubcoreMesh(core_axis_name="c", subcore_axis_name="s"),
        out_type=jax.ShapeDtypeStruct((N, D), jnp.float32),
        scratch_types=[
            pltpu.VMEM((NCHUNK, GCHUNK), jnp.int32),
            pltpu.VMEM((ROWS_PER_W, D), jnp.float32),
            pltpu.SemaphoreType.DMA,
        ],
    )
    def _sc_gather(codebook_hbm, idx_hbm, out_hbm, idx_v, rows_v, sem):
        wid = lax.axis_index("s") * 2 + lax.axis_index("c")
        base = wid * ROWS_PER_W
        pltpu.sync_copy(idx_hbm.at[wid], idx_v)
        copies = []
        for j in range(NCHUNK):
            copies.append(pltpu.async_copy(
                codebook_hbm.at[idx_v.at[j]],
                rows_v.at[pl.ds(j * GCHUNK, GCHUNK)],
                sem,
            ))
        for c in copies:
            c.wait()
        pltpu.sync_copy(rows_v, out_hbm.at[pl.ds(base, ROWS_PER_W)])

    return _sc_gather


def _loss_body(x_ref, q_ref, idx_ref, qst_ref, vq_ref, com_ref, cb_ref,
               perp_ref, acc_ref, counts_ref):
    i = pl.program_id(0)

    @pl.when(i == 0)
    def _init():
        acc_ref[0] = 0.0
        acc_ref[1] = 0.0
        counts_ref[...] = jnp.zeros((1, K), jnp.float32)

    x = x_ref[...]
    q = q_ref[...]
    qst = x + (q - x)
    qst_ref[...] = qst
    de = qst - x
    dq = q - x
    acc_ref[0] += jnp.sum(de * de)
    acc_ref[1] += jnp.sum(dq * dq)
    ii = lax.broadcasted_iota(jnp.int32, (BLK, K), 1)
    onehot = (idx_ref[...] == ii).astype(jnp.float32)
    counts_ref[...] += jnp.sum(onehot, axis=0, keepdims=True)

    @pl.when(i == NBLK - 1)
    def _final():
        e_loss = acc_ref[0] / (N * D)
        q_loss = acc_ref[1] / (N * D)
        commitment = COMMIT * e_loss
        avg = counts_ref[...] / N
        ent = avg * jnp.log(avg + 1e-10)
        vq_ref[0, 0] = commitment + q_loss
        com_ref[0, 0] = commitment
        cb_ref[0, 0] = q_loss
        perp_ref[0, 0] = jnp.exp(-jnp.sum(ent))


_loss_call = pl.pallas_call(
    _loss_body,
    grid=(NBLK,),
    in_specs=[
        pl.BlockSpec((BLK, D), lambda i: (i, 0)),
        pl.BlockSpec((BLK, D), lambda i: (i, 0)),
        pl.BlockSpec((BLK, 1), lambda i: (i, 0)),
    ],
    out_specs=[
        pl.BlockSpec((BLK, D), lambda i: (i, 0)),
        pl.BlockSpec((1, 1), lambda i: (0, 0), memory_space=pltpu.SMEM),
        pl.BlockSpec((1, 1), lambda i: (0, 0), memory_space=pltpu.SMEM),
        pl.BlockSpec((1, 1), lambda i: (0, 0), memory_space=pltpu.SMEM),
        pl.BlockSpec((1, 1), lambda i: (0, 0), memory_space=pltpu.SMEM),
    ],
    out_shape=[
        jax.ShapeDtypeStruct((N, D), jnp.float32),
        jax.ShapeDtypeStruct((1, 1), jnp.float32),
        jax.ShapeDtypeStruct((1, 1), jnp.float32),
        jax.ShapeDtypeStruct((1, 1), jnp.float32),
        jax.ShapeDtypeStruct((1, 1), jnp.float32),
    ],
    scratch_shapes=[
        pltpu.SMEM((2,), jnp.float32),
        pltpu.VMEM((1, K), jnp.float32),
    ],
)


def kernel(inputs, codebook):
    shape = inputs.shape
    flat = inputs.reshape(N, D)
    ct = codebook.T
    sx = jnp.sum(flat**2, axis=1)[:, None]
    sc = jnp.sum(codebook**2, axis=1)[None, :]
    idx2d = _argmin_call(flat, ct, sx, sc)
    idx_sc = idx2d.reshape(NWORK, NCHUNK, GCHUNK)
    quantized = _sc_gather_call()(codebook, idx_sc)
    qst, vq, com, cb, perp = _loss_call(flat, quantized, idx2d)
    return (
        qst.reshape(shape),
        idx2d.reshape(shape[:-1]),
        vq[0, 0],
        com[0, 0],
        cb[0, 0],
        perp[0, 0],
    )


# speedup vs baseline: 1.6820x; 1.6820x over previous
import functools

import jax
import jax.numpy as jnp
from jax import lax
from jax.experimental import pallas as pl
from jax.experimental.pallas import tpu as pltpu
from jax.experimental.pallas import tpu_sc as plsc

N = 9216
K = 8192
D = 256
BLK = 512
NBLK = N // BLK
NWORK = 32
ROWS_PER_W = N // NWORK
GCHUNK = 96
NCHUNK = ROWS_PER_W // GCHUNK
COMMIT = 0.25


_SEG_BOUNDS = (0, 2736, 5472, K)


def _argmin_body(x_ref, ct_ref, sx_ref, sc_ref, idx_ref):
    x = x_ref[...]
    ct = ct_ref[...]
    m = jnp.dot(x.astype(jnp.bfloat16), ct.astype(jnp.bfloat16),
                preferred_element_type=jnp.float32)
    dist = (sx_ref[...] + sc_ref[...]) - 2.0 * m
    ii = lax.broadcasted_iota(jnp.int32, (BLK, K), 1)
    inf = jnp.float32(jnp.inf)
    acc_v = jnp.full((BLK,), inf, jnp.float32)
    acc_i = jnp.zeros((BLK,), jnp.int32)
    for a, b in zip(_SEG_BOUNDS[:-1], _SEG_BOUNDS[1:]):
        seg = jnp.where((ii >= a) & (ii < b), dist, inf)
        wv = jnp.min(seg, axis=1)
        wi = jnp.min(jnp.where(seg == wv[:, None], ii, K), axis=1)
        keep = (acc_v < wv) | ((acc_v == wv) & (acc_i < wi))
        acc_v = jnp.where(keep, acc_v, wv)
        acc_i = jnp.where(keep, acc_i, wi)
        acc_v = acc_v.astype(jnp.bfloat16).astype(jnp.float32)
    idx_ref[...] = acc_i[:, None]


_argmin_call = pl.pallas_call(
    _argmin_body,
    grid=(NBLK,),
    in_specs=[
        pl.BlockSpec((BLK, D), lambda i: (i, 0)),
        pl.BlockSpec((D, K), lambda i: (0, 0)),
        pl.BlockSpec((BLK, 1), lambda i: (i, 0)),
        pl.BlockSpec((1, K), lambda i: (0, 0)),
    ],
    out_specs=pl.BlockSpec((BLK, 1), lambda i: (i, 0)),
    out_shape=jax.ShapeDtypeStruct((N, 1), jnp.int32),
)


@functools.cache
def _sc_gather_call():
    @functools.partial(
        pl.kernel,
        mesh=plsc.VectorSubcoreMesh(core_axis_name="c", subcore_axis_name="s"),
        out_type=jax.ShapeDtypeStruct((N, D), jnp.float32),
        scratch_types=[
            pltpu.VMEM((NCHUNK, GCHUNK), jnp.int32),
            pltpu.VMEM((ROWS_PER_W, D), jnp.float32),
            pltpu.SemaphoreType.DMA,
        ],
    )
    def _sc_gather(codebook_hbm, idx_hbm, out_hbm, idx_v, rows_v, sem):
        wid = lax.axis_index("s") * 2 + lax.axis_index("c")
        base = wid * ROWS_PER_W
        pltpu.sync_copy(idx_hbm.at[wid], idx_v)
        copies = []
        for j in range(NCHUNK):
            copies.append(pltpu.async_copy(
                codebook_hbm.at[idx_v.at[j]],
                rows_v.at[pl.ds(j * GCHUNK, GCHUNK)],
                sem,
            ))
        for c in copies:
            c.wait()
        pltpu.sync_copy(rows_v, out_hbm.at[pl.ds(base, ROWS_PER_W)])

    return _sc_gather


def _loss_body(x_ref, q_ref, idx_ref, qst_ref, vq_ref, com_ref, cb_ref,
               perp_ref, acc_ref, counts_ref):
    i = pl.program_id(0)

    @pl.when(i == 0)
    def _init():
        acc_ref[0] = 0.0
        acc_ref[1] = 0.0
        counts_ref[...] = jnp.zeros((1, K), jnp.float32)

    x = x_ref[...]
    q = q_ref[...]
    qst = x + (q - x)
    qst_ref[...] = qst
    de = qst - x
    dq = q - x
    acc_ref[0] += jnp.sum(de * de)
    acc_ref[1] += jnp.sum(dq * dq)
    ii = lax.broadcasted_iota(jnp.int32, (BLK, K), 1)
    onehot = (idx_ref[...] == ii).astype(jnp.float32)
    counts_ref[...] += jnp.sum(onehot, axis=0, keepdims=True)

    @pl.when(i == NBLK - 1)
    def _final():
        e_loss = acc_ref[0] / (N * D)
        q_loss = acc_ref[1] / (N * D)
        commitment = COMMIT * e_loss
        avg = counts_ref[...] / N
        ent = avg * jnp.log(avg + 1e-10)
        vq_ref[0, 0] = commitment + q_loss
        com_ref[0, 0] = commitment
        cb_ref[0, 0] = q_loss
        perp_ref[0, 0] = jnp.exp(-jnp.sum(ent))


_loss_call = pl.pallas_call(
    _loss_body,
    grid=(NBLK,),
    in_specs=[
        pl.BlockSpec((BLK, D), lambda i: (i, 0)),
        pl.BlockSpec((BLK, D), lambda i: (i, 0)),
        pl.BlockSpec((BLK, 1), lambda i: (i, 0)),
    ],
    out_specs=[
        pl.BlockSpec((BLK, D), lambda i: (i, 0)),
        pl.BlockSpec((1, 1), lambda i: (0, 0), memory_space=pltpu.SMEM),
        pl.BlockSpec((1, 1), lambda i: (0, 0), memory_space=pltpu.SMEM),
        pl.BlockSpec((1, 1), lambda i: (0, 0), memory_space=pltpu.SMEM),
        pl.BlockSpec((1, 1), lambda i: (0, 0), memory_space=pltpu.SMEM),
    ],
    out_shape=[
        jax.ShapeDtypeStruct((N, D), jnp.float32),
        jax.ShapeDtypeStruct((1, 1), jnp.float32),
        jax.ShapeDtypeStruct((1, 1), jnp.float32),
        jax.ShapeDtypeStruct((1, 1), jnp.float32),
        jax.ShapeDtypeStruct((1, 1), jnp.float32),
    ],
    scratch_shapes=[
        pltpu.SMEM((2,), jnp.float32),
        pltpu.VMEM((1, K), jnp.float32),
    ],
)


def kernel(inputs, codebook):
    shape = inputs.shape
    flat = inputs.reshape(N, D)
    ct = codebook.T
    sx = jnp.sum(flat**2, axis=1)[:, None]
    sc = jnp.sum(codebook**2, axis=1)[None, :]
    idx2d = _argmin_call(flat, ct, sx, sc)
    idx_sc = idx2d.reshape(NWORK, NCHUNK, GCHUNK)
    quantized = _sc_gather_call()(codebook, idx_sc)
    qst, vq, com, cb, perp = _loss_call(flat, quantized, idx2d)
    return (
        qst.reshape(shape),
        idx2d.reshape(shape[:-1]),
        vq[0, 0],
        com[0, 0],
        cb[0, 0],
        perp[0, 0],
    )
